# baseline (speedup 1.0000x reference)
import functools

import jax
import jax.numpy as jnp
from jax import lax
from jax.experimental import pallas as pl
from jax.experimental.pallas import tpu as pltpu
from jax.experimental.pallas import tpu_sc as plsc

_N = 10000
_D = 128
_E = 320000
_NC = 2
_NS = 16
_L = 16
_K = 80
_EPT = _E // (_NC * _NS)
_NCH = _EPT // _K
_RSTRIDE = 624
_RWIN = 640


def _prologue_body(src_ref, dst_ref, wt_ref, bb_ref, s1_ref, s2_ref):
    w = wt_ref[...]
    w1 = w[:, :_D]
    w2 = w[:, _D:]
    s1_ref[...] = jnp.sum(src_ref[...] * w1, axis=1, keepdims=True)
    s2_ref[...] = jnp.sum(dst_ref[...] * w2, axis=1, keepdims=True) + bb_ref[...]


def _epilogue_body(aggp_ref, cntp_ref, zr_ref, out_ref):
    zr = zr_ref[...]
    agg = aggp_ref[0] + aggp_ref[1] + zr
    cnt = cntp_ref[0, :, 0:1] + cntp_ref[1, :, 0:1] + zr
    cnt = jnp.clip(cnt, 1e-8, None)
    out_ref[...] = agg / cnt


def _sc_body(srcf, s1h, s2h, eidx_h, aggo, cnto,
             agg_spm, cnt_spm,
             eidx, rows, cntr, s1g, s2g, attv,
             isem0, isem1, isem2, isem3,
             gsem0, gsem1, gsem2, ssem0, ssem1, ssem2):
    c = lax.axis_index("c")
    s = lax.axis_index("s")
    tile_base = (c * _NS + s) * _EPT
    row_base = s * _RSTRIDE
    isems = (isem0, isem1, isem2, isem3)
    gsems = (gsem0, gsem1, gsem2)
    ssems = (ssem0, ssem1, ssem2)

    zero16 = jnp.zeros((_L,), jnp.float32)

    def _zs(i, carry):
        r = i // 8
        q = lax.rem(i, 8)
        rows[0, r, pl.ds(q * _L, _L)] = zero16
        return carry

    lax.fori_loop(0, _K * 8, _zs, 0)

    def _zc(i, carry):
        r = i // _K
        q = lax.rem(i, _K)
        cntr[r, q, :] = zero16
        return carry

    lax.fori_loop(0, 3 * _K, _zc, 0)

    off = 0
    while off < _RWIN:
        sz = min(_K, _RWIN - off)
        pltpu.async_copy(rows.at[0, pl.ds(0, sz)],
                         agg_spm.at[pl.ds(row_base + off, sz)], isem0)
        pltpu.async_copy(cntr.at[0, pl.ds(0, sz)],
                         cnt_spm.at[pl.ds(row_base + off, sz)], isem1)
        off += sz
    off = 0
    while off < _RWIN:
        sz = min(_K, _RWIN - off)
        pltpu.make_async_copy(rows.at[0, pl.ds(0, sz)],
                              agg_spm.at[pl.ds(row_base + off, sz)],
                              isem0).wait()
        pltpu.make_async_copy(cntr.at[0, pl.ds(0, sz)],
                              cnt_spm.at[pl.ds(row_base + off, sz)],
                              isem1).wait()
        off += sz
    plsc.subcore_barrier()

    def _idx_load(g, i):
        eb = tile_base + g * _K
        pltpu.async_copy(eidx_h.at[:, pl.ds(eb, _K)], eidx.at[i], isems[i])

    def _wait_idx(g, i):
        eb = tile_base + g * _K
        pltpu.make_async_copy(eidx_h.at[:, pl.ds(eb, _K)], eidx.at[i],
                              isems[i]).wait()

    def _gather(r, i):
        pltpu.async_copy(srcf.at[eidx.at[i, 0]], rows.at[r], gsems[r])
        pltpu.async_copy(s1h.at[eidx.at[i, 0]], s1g.at[r], gsems[r])
        pltpu.async_copy(s2h.at[eidx.at[i, 1]], s2g.at[r], gsems[r])

    def _wait_gather(r, i):
        pltpu.make_async_copy(srcf.at[eidx.at[i, 0]], rows.at[r],
                              gsems[r]).wait()
        pltpu.make_async_copy(s1h.at[eidx.at[i, 0]], s1g.at[r],
                              gsems[r]).wait()
        pltpu.make_async_copy(s2h.at[eidx.at[i, 1]], s2g.at[r],
                              gsems[r]).wait()

    lane = lax.iota(jnp.int32, _L)
    zlane = jnp.zeros((_L,), jnp.int32)

    def _compute(r):
        for v in range(_K // _L):
            sl = pl.ds(v * _L, _L)
            att = 1.0 / (1.0 + jnp.exp(-(s1g[r, sl] + s2g[r, sl])))
            attv[sl] = att
            plsc.store_scatter(cntr.at[r], [lane + v * _L, zlane], att)

        @plsc.parallel_loop(0, _K, step=1, unroll=8)
        def _edge(e):
            bc = plsc.load_gather(attv, [jnp.full((_L,), 0, jnp.int32) + e])
            for dd in range(_D // _L):
                dsl = pl.ds(dd * _L, _L)
                rows[r, e, dsl] = rows[r, e, dsl] * bc

    def _scatter(r, i):
        pltpu.async_copy(rows.at[r], agg_spm.at[eidx.at[i, 1]], ssems[r],
                         add=True)
        pltpu.async_copy(cntr.at[r], cnt_spm.at[eidx.at[i, 1]], ssems[r],
                         add=True)

    def _wait_scatter(r, i):
        pltpu.make_async_copy(rows.at[r], agg_spm.at[eidx.at[i, 1]],
                              ssems[r]).wait()
        pltpu.make_async_copy(cntr.at[r], cnt_spm.at[eidx.at[i, 1]],
                              ssems[r]).wait()

    def _pstep(g, gm3, gm4, p1, p2, p3):
        if p1:
            _wait_scatter((gm3 + 1) % 3, (gm4 + 2) % 4)
        if p2:
            _idx_load(g + 2, (gm4 + 2) % 4)
        if p3:
            _wait_idx(g + 1, (gm4 + 1) % 4)
            _gather((gm3 + 1) % 3, (gm4 + 1) % 4)
        _wait_gather(gm3, gm4)
        _compute(gm3)
        _scatter(gm3, gm4)

    _idx_load(0, 0)
    _idx_load(1, 1)
    _wait_idx(0, 0)
    _gather(0, 0)
    _pstep(0, 0, 0, False, True, True)
    _pstep(1, 1, 1, False, True, True)

    def _twelve(t, carry):
        for j in range(12):
            g = 12 * t + 2 + j
            _pstep(g, (2 + j) % 3, (2 + j) % 4, True, True, True)
        return carry

    lax.fori_loop(0, (_NCH - 5) // 12, _twelve, 0)

    _pstep(_NCH - 3, (_NCH - 3) % 3, (_NCH - 3) % 4, True, True, True)
    _pstep(_NCH - 2, (_NCH - 2) % 3, (_NCH - 2) % 4, True, False, True)
    _pstep(_NCH - 1, (_NCH - 1) % 3, (_NCH - 1) % 4, True, False, False)
    _wait_scatter((_NCH - 2) % 3, (_NCH - 2) % 4)
    _wait_scatter((_NCH - 1) % 3, (_NCH - 1) % 4)
    plsc.subcore_barrier()

    pltpu.sync_copy(agg_spm.at[pl.ds(row_base, _RWIN)],
                    aggo.at[c, pl.ds(row_base, _RWIN)])
    pltpu.sync_copy(cnt_spm.at[pl.ds(row_base, _RWIN)],
                    cnto.at[c, pl.ds(row_base, _RWIN)])


@functools.cache
def _make_sc_kernel():
    return functools.partial(
        pl.kernel,
        mesh=plsc.VectorSubcoreMesh(core_axis_name="c", subcore_axis_name="s",
                                    num_cores=_NC, num_subcores=_NS),
        out_type=[
            jax.ShapeDtypeStruct((_NC, _N, _D), jnp.float32),
            jax.ShapeDtypeStruct((_NC, _N, _L), jnp.float32),
        ],
        scratch_types=[
            pltpu.VMEM_SHARED((_N, _D), jnp.float32),
            pltpu.VMEM_SHARED((_N, _L), jnp.float32),
            pltpu.VMEM((4, 2, _K), jnp.int32),
            pltpu.VMEM((3, _K, _D), jnp.float32),
            pltpu.VMEM((3, _K, _L), jnp.float32),
            pltpu.VMEM((3, _K), jnp.float32),
            pltpu.VMEM((3, _K), jnp.float32),
            pltpu.VMEM((_K,), jnp.float32),
            pltpu.SemaphoreType.DMA,
            pltpu.SemaphoreType.DMA,
            pltpu.SemaphoreType.DMA,
            pltpu.SemaphoreType.DMA,
            pltpu.SemaphoreType.DMA,
            pltpu.SemaphoreType.DMA,
            pltpu.SemaphoreType.DMA,
            pltpu.SemaphoreType.DMA,
            pltpu.SemaphoreType.DMA,
            pltpu.SemaphoreType.DMA,
        ],
        compiler_params=pltpu.CompilerParams(use_tc_tiling_on_sc=False,
                                             needs_layout_passes=False),
    )(_sc_body)


def kernel(src_feat, dst_feat, edge_index, n_dst, W, b):
    wt = W.reshape(1, 2 * _D)
    bb = b.reshape(1, 1)
    s1, s2 = pl.pallas_call(
        _prologue_body,
        out_shape=[
            jax.ShapeDtypeStruct((_N, 1), jnp.float32),
            jax.ShapeDtypeStruct((_N, 1), jnp.float32),
        ],
    )(src_feat, dst_feat, wt, bb)
    s1 = s1.reshape(_N)
    s2 = s2.reshape(_N)
    aggp, cntp = _make_sc_kernel()(src_feat, s1, s2, edge_index)
    zr = (jnp.asarray(n_dst, jnp.float32) - jnp.float32(_N)).reshape(1, 1)
    out = pl.pallas_call(
        _epilogue_body,
        out_shape=jax.ShapeDtypeStruct((_N, _D), jnp.float32),
    )(aggp, cntp, zr)
    return out


# speedup vs baseline: 1.0298x; 1.0298x over previous
import functools

import jax
import jax.numpy as jnp
from jax import lax
from jax.experimental import pallas as pl
from jax.experimental.pallas import tpu as pltpu
from jax.experimental.pallas import tpu_sc as plsc

_N = 10000
_D = 128
_E = 320000
_NC = 2
_NS = 16
_L = 16
_K = 80
_EPT = _E // (_NC * _NS)
_NCH = _EPT // _K
_RSTRIDE = 624
_RWIN = 640


def _prologue_body(src_ref, dst_ref, wt_ref, bb_ref, s1_ref, s2_ref):
    w = wt_ref[...]
    w1 = w[:, :_D]
    w2 = w[:, _D:]
    s1_ref[...] = jnp.sum(src_ref[...] * w1, axis=1, keepdims=True)
    s2_ref[...] = jnp.sum(dst_ref[...] * w2, axis=1, keepdims=True) + bb_ref[...]


def _epilogue_body(aggp_ref, cntp_ref, zr_ref, out_ref):
    zr = zr_ref[...]
    agg = aggp_ref[0] + aggp_ref[1] + zr
    cnt = cntp_ref[0, :, 0:1] + cntp_ref[1, :, 0:1] + zr
    cnt = jnp.clip(cnt, 1e-8, None)
    out_ref[...] = agg / cnt


def _sc_body(srcf, s1h, s2h, eidx_h, aggo, cnto,
             agg_spm, cnt_spm,
             eidx, rows, cntr, s1g, s2g, attv,
             isem0, isem1, isem2, isem3,
             gsem0, gsem1, gsem2, ssem0, ssem1, ssem2):
    c = lax.axis_index("c")
    s = lax.axis_index("s")
    tile_base = (c * _NS + s) * _EPT
    row_base = s * _RSTRIDE
    isems = (isem0, isem1, isem2, isem3)
    gsems = (gsem0, gsem1, gsem2)
    ssems = (ssem0, ssem1, ssem2)

    zero16 = jnp.zeros((_L,), jnp.float32)

    def _zs(i, carry):
        r = i // 8
        q = lax.rem(i, 8)
        rows[0, r, pl.ds(q * _L, _L)] = zero16
        return carry

    lax.fori_loop(0, _K * 8, _zs, 0)

    def _zc(i, carry):
        r = i // _K
        q = lax.rem(i, _K)
        cntr[r, q, :] = zero16
        return carry

    lax.fori_loop(0, 3 * _K, _zc, 0)

    off = 0
    while off < _RWIN:
        sz = min(_K, _RWIN - off)
        pltpu.async_copy(rows.at[0, pl.ds(0, sz)],
                         agg_spm.at[pl.ds(row_base + off, sz)], isem0)
        pltpu.async_copy(cntr.at[0, pl.ds(0, sz)],
                         cnt_spm.at[pl.ds(row_base + off, sz)], isem1)
        off += sz
    off = 0
    while off < _RWIN:
        sz = min(_K, _RWIN - off)
        pltpu.make_async_copy(rows.at[0, pl.ds(0, sz)],
                              agg_spm.at[pl.ds(row_base + off, sz)],
                              isem0).wait()
        pltpu.make_async_copy(cntr.at[0, pl.ds(0, sz)],
                              cnt_spm.at[pl.ds(row_base + off, sz)],
                              isem1).wait()
        off += sz
    plsc.subcore_barrier()

    def _idx_load(g, i):
        eb = tile_base + g * _K
        pltpu.async_copy(eidx_h.at[:, pl.ds(eb, _K)], eidx.at[i], isems[i])

    def _wait_idx(g, i):
        eb = tile_base + g * _K
        pltpu.make_async_copy(eidx_h.at[:, pl.ds(eb, _K)], eidx.at[i],
                              isems[i]).wait()

    def _gather(r, i):
        pltpu.async_copy(srcf.at[eidx.at[i, 0]], rows.at[r], gsems[r])
        pltpu.async_copy(s1h.at[eidx.at[i, 0]], s1g.at[r], gsems[r])
        pltpu.async_copy(s2h.at[eidx.at[i, 1]], s2g.at[r], gsems[r])

    def _wait_gather(r, i):
        pltpu.make_async_copy(srcf.at[eidx.at[i, 0]], rows.at[r],
                              gsems[r]).wait()
        pltpu.make_async_copy(s1h.at[eidx.at[i, 0]], s1g.at[r],
                              gsems[r]).wait()
        pltpu.make_async_copy(s2h.at[eidx.at[i, 1]], s2g.at[r],
                              gsems[r]).wait()

    lane = lax.iota(jnp.int32, _L)
    zlane = jnp.zeros((_L,), jnp.int32)

    def _compute(r):
        for v in range(_K // _L):
            sl = pl.ds(v * _L, _L)
            att = 1.0 / (1.0 + jnp.exp(-(s1g[r, sl] + s2g[r, sl])))
            attv[sl] = att
            plsc.store_scatter(cntr.at[r], [lane + v * _L, zlane], att)

        @plsc.parallel_loop(0, _K, step=1, unroll=4)
        def _edge(e):
            bc = plsc.load_gather(attv, [jnp.full((_L,), 0, jnp.int32) + e])
            for dd in range(_D // _L):
                dsl = pl.ds(dd * _L, _L)
                rows[r, e, dsl] = rows[r, e, dsl] * bc

    def _scatter(r, i):
        pltpu.async_copy(rows.at[r], agg_spm.at[eidx.at[i, 1]], ssems[r],
                         add=True)
        pltpu.async_copy(cntr.at[r], cnt_spm.at[eidx.at[i, 1]], ssems[r],
                         add=True)

    def _wait_scatter(r, i):
        pltpu.make_async_copy(rows.at[r], agg_spm.at[eidx.at[i, 1]],
                              ssems[r]).wait()
        pltpu.make_async_copy(cntr.at[r], cnt_spm.at[eidx.at[i, 1]],
                              ssems[r]).wait()

    def _pstep(g, gm3, gm4, p1, p2, p3):
        if p1:
            _wait_scatter((gm3 + 1) % 3, (gm4 + 2) % 4)
        if p2:
            _idx_load(g + 2, (gm4 + 2) % 4)
        if p3:
            _wait_idx(g + 1, (gm4 + 1) % 4)
            _gather((gm3 + 1) % 3, (gm4 + 1) % 4)
        _wait_gather(gm3, gm4)
        _compute(gm3)
        _scatter(gm3, gm4)

    _idx_load(0, 0)
    _idx_load(1, 1)
    _wait_idx(0, 0)
    _gather(0, 0)
    _pstep(0, 0, 0, False, True, True)
    _pstep(1, 1, 1, False, True, True)

    def _twelve(t, carry):
        for j in range(12):
            g = 12 * t + 2 + j
            _pstep(g, (2 + j) % 3, (2 + j) % 4, True, True, True)
        return carry

    lax.fori_loop(0, (_NCH - 5) // 12, _twelve, 0)

    _pstep(_NCH - 3, (_NCH - 3) % 3, (_NCH - 3) % 4, True, True, True)
    _pstep(_NCH - 2, (_NCH - 2) % 3, (_NCH - 2) % 4, True, False, True)
    _pstep(_NCH - 1, (_NCH - 1) % 3, (_NCH - 1) % 4, True, False, False)
    _wait_scatter((_NCH - 2) % 3, (_NCH - 2) % 4)
    _wait_scatter((_NCH - 1) % 3, (_NCH - 1) % 4)
    plsc.subcore_barrier()

    pltpu.sync_copy(agg_spm.at[pl.ds(row_base, _RWIN)],
                    aggo.at[c, pl.ds(row_base, _RWIN)])
    pltpu.sync_copy(cnt_spm.at[pl.ds(row_base, _RWIN)],
                    cnto.at[c, pl.ds(row_base, _RWIN)])


@functools.cache
def _make_sc_kernel():
    return functools.partial(
        pl.kernel,
        mesh=plsc.VectorSubcoreMesh(core_axis_name="c", subcore_axis_name="s",
                                    num_cores=_NC, num_subcores=_NS),
        out_type=[
            jax.ShapeDtypeStruct((_NC, _N, _D), jnp.float32),
            jax.ShapeDtypeStruct((_NC, _N, _L), jnp.float32),
        ],
        scratch_types=[
            pltpu.VMEM_SHARED((_N, _D), jnp.float32),
            pltpu.VMEM_SHARED((_N, _L), jnp.float32),
            pltpu.VMEM((4, 2, _K), jnp.int32),
            pltpu.VMEM((3, _K, _D), jnp.float32),
            pltpu.VMEM((3, _K, _L), jnp.float32),
            pltpu.VMEM((3, _K), jnp.float32),
            pltpu.VMEM((3, _K), jnp.float32),
            pltpu.VMEM((_K,), jnp.float32),
            pltpu.SemaphoreType.DMA,
            pltpu.SemaphoreType.DMA,
            pltpu.SemaphoreType.DMA,
            pltpu.SemaphoreType.DMA,
            pltpu.SemaphoreType.DMA,
            pltpu.SemaphoreType.DMA,
            pltpu.SemaphoreType.DMA,
            pltpu.SemaphoreType.DMA,
            pltpu.SemaphoreType.DMA,
            pltpu.SemaphoreType.DMA,
        ],
        compiler_params=pltpu.CompilerParams(use_tc_tiling_on_sc=False,
                                             needs_layout_passes=False),
    )(_sc_body)


def kernel(src_feat, dst_feat, edge_index, n_dst, W, b):
    wt = W.reshape(1, 2 * _D)
    bb = b.reshape(1, 1)
    s1, s2 = pl.pallas_call(
        _prologue_body,
        out_shape=[
            jax.ShapeDtypeStruct((_N, 1), jnp.float32),
            jax.ShapeDtypeStruct((_N, 1), jnp.float32),
        ],
    )(src_feat, dst_feat, wt, bb)
    s1 = s1.reshape(_N)
    s2 = s2.reshape(_N)
    aggp, cntp = _make_sc_kernel()(src_feat, s1, s2, edge_index)
    zr = (jnp.asarray(n_dst, jnp.float32) - jnp.float32(_N)).reshape(1, 1)
    out = pl.pallas_call(
        _epilogue_body,
        out_shape=jax.ShapeDtypeStruct((_N, _D), jnp.float32),
    )(aggp, cntp, zr)
    return out


# speedup vs baseline: 1.0448x; 1.0146x over previous
import functools

import jax
import jax.numpy as jnp
from jax import lax
from jax.experimental import pallas as pl
from jax.experimental.pallas import tpu as pltpu
from jax.experimental.pallas import tpu_sc as plsc

_N = 10000
_D = 128
_E = 320000
_NC = 2
_NS = 16
_L = 16
_K = 80
_EPT = _E // (_NC * _NS)
_NCH = _EPT // _K
_RSTRIDE = 624
_RWIN = 640


def _prologue_body(src_ref, dst_ref, wt_ref, bb_ref, s1_ref, s2_ref):
    w = wt_ref[...]
    w1 = w[:, :_D]
    w2 = w[:, _D:]
    s1_ref[...] = jnp.sum(src_ref[...] * w1, axis=1, keepdims=True)
    s2_ref[...] = jnp.sum(dst_ref[...] * w2, axis=1, keepdims=True) + bb_ref[...]


def _epilogue_body(aggp_ref, cntp_ref, zr_ref, out_ref):
    zr = zr_ref[...]
    agg = aggp_ref[0] + aggp_ref[1] + zr
    cnt = cntp_ref[0, :, 0:1] + cntp_ref[1, :, 0:1] + zr
    cnt = jnp.clip(cnt, 1e-8, None)
    out_ref[...] = agg / cnt


def _sc_body(srcf, s1h, s2h, eidx_h, aggo, cnto,
             agg_spm, cnt_spm,
             eidx, rows, cntr, s1g, s2g, attv,
             isem0, isem1, isem2, isem3,
             gsem0, gsem1, gsem2, ssem0, ssem1, ssem2):
    c = lax.axis_index("c")
    s = lax.axis_index("s")
    tile_base = (c * _NS + s) * _EPT
    row_base = s * _RSTRIDE
    isems = (isem0, isem1, isem2, isem3)
    gsems = (gsem0, gsem1, gsem2)
    ssems = (ssem0, ssem1, ssem2)

    zero16 = jnp.zeros((_L,), jnp.float32)

    def _zs(i, carry):
        r = i // 8
        q = lax.rem(i, 8)
        rows[0, r, pl.ds(q * _L, _L)] = zero16
        return carry

    lax.fori_loop(0, _K * 8, _zs, 0)

    def _zc(i, carry):
        cntr[0, i, :] = zero16
        return carry

    lax.fori_loop(0, _K, _zc, 0)

    off = 0
    while off < _RWIN:
        sz = min(_K, _RWIN - off)
        pltpu.sync_copy(rows.at[0, pl.ds(0, sz)],
                        agg_spm.at[pl.ds(row_base + off, sz)])
        pltpu.sync_copy(cntr.at[0, pl.ds(0, sz)],
                        cnt_spm.at[pl.ds(row_base + off, sz)])
        off += sz
    plsc.subcore_barrier()

    def _idx_load(g, i):
        eb = tile_base + g * _K
        pltpu.async_copy(eidx_h.at[:, pl.ds(eb, _K)], eidx.at[i], isems[i])

    def _wait_idx(g, i):
        eb = tile_base + g * _K
        pltpu.make_async_copy(eidx_h.at[:, pl.ds(eb, _K)], eidx.at[i],
                              isems[i]).wait()

    def _gather(r, i):
        pltpu.async_copy(srcf.at[eidx.at[i, 0]], rows.at[r], gsems[r])
        pltpu.async_copy(s1h.at[eidx.at[i, 0]], s1g.at[r], gsems[r])
        pltpu.async_copy(s2h.at[eidx.at[i, 1]], s2g.at[r], gsems[r])

    def _wait_gather(r, i):
        pltpu.make_async_copy(srcf.at[eidx.at[i, 0]], rows.at[r],
                              gsems[r]).wait()
        pltpu.make_async_copy(s1h.at[eidx.at[i, 0]], s1g.at[r],
                              gsems[r]).wait()
        pltpu.make_async_copy(s2h.at[eidx.at[i, 1]], s2g.at[r],
                              gsems[r]).wait()

    def _compute(r):
        for v in range(_K // _L):
            sl = pl.ds(v * _L, _L)
            att = 1.0 / (1.0 + jnp.exp(-(s1g[r, sl] + s2g[r, sl])))
            attv[sl] = att

        @plsc.parallel_loop(0, _K, step=1, unroll=4)
        def _edge(e):
            bc = plsc.load_gather(attv, [jnp.full((_L,), 0, jnp.int32) + e])
            cntr[r, e, :] = bc
            for dd in range(_D // _L):
                dsl = pl.ds(dd * _L, _L)
                rows[r, e, dsl] = rows[r, e, dsl] * bc

    def _scatter(r, i):
        pltpu.async_copy(rows.at[r], agg_spm.at[eidx.at[i, 1]], ssems[r],
                         add=True)
        pltpu.async_copy(cntr.at[r], cnt_spm.at[eidx.at[i, 1]], ssems[r],
                         add=True)

    def _wait_scatter(r, i):
        pltpu.make_async_copy(rows.at[r], agg_spm.at[eidx.at[i, 1]],
                              ssems[r]).wait()
        pltpu.make_async_copy(cntr.at[r], cnt_spm.at[eidx.at[i, 1]],
                              ssems[r]).wait()

    def _pstep(g, gm3, gm4, p1, p2, p3):
        if p1:
            _wait_scatter((gm3 + 1) % 3, (gm4 + 2) % 4)
        if p2:
            _idx_load(g + 2, (gm4 + 2) % 4)
        if p3:
            _wait_idx(g + 1, (gm4 + 1) % 4)
            _gather((gm3 + 1) % 3, (gm4 + 1) % 4)
        _wait_gather(gm3, gm4)
        _compute(gm3)
        _scatter(gm3, gm4)

    _idx_load(0, 0)
    _idx_load(1, 1)
    _wait_idx(0, 0)
    _gather(0, 0)
    _pstep(0, 0, 0, False, True, True)
    _pstep(1, 1, 1, False, True, True)

    def _twelve(t, carry):
        for j in range(12):
            g = 12 * t + 2 + j
            _pstep(g, (2 + j) % 3, (2 + j) % 4, True, True, True)
        return carry

    lax.fori_loop(0, (_NCH - 5) // 12, _twelve, 0)

    _pstep(_NCH - 3, (_NCH - 3) % 3, (_NCH - 3) % 4, True, True, True)
    _pstep(_NCH - 2, (_NCH - 2) % 3, (_NCH - 2) % 4, True, False, True)
    _pstep(_NCH - 1, (_NCH - 1) % 3, (_NCH - 1) % 4, True, False, False)
    _wait_scatter((_NCH - 2) % 3, (_NCH - 2) % 4)
    _wait_scatter((_NCH - 1) % 3, (_NCH - 1) % 4)
    plsc.subcore_barrier()

    pltpu.sync_copy(agg_spm.at[pl.ds(row_base, _RWIN)],
                    aggo.at[c, pl.ds(row_base, _RWIN)])
    pltpu.sync_copy(cnt_spm.at[pl.ds(row_base, _RWIN)],
                    cnto.at[c, pl.ds(row_base, _RWIN)])


@functools.cache
def _make_sc_kernel():
    return functools.partial(
        pl.kernel,
        mesh=plsc.VectorSubcoreMesh(core_axis_name="c", subcore_axis_name="s",
                                    num_cores=_NC, num_subcores=_NS),
        out_type=[
            jax.ShapeDtypeStruct((_NC, _N, _D), jnp.float32),
            jax.ShapeDtypeStruct((_NC, _N, _L), jnp.float32),
        ],
        scratch_types=[
            pltpu.VMEM_SHARED((_N, _D), jnp.float32),
            pltpu.VMEM_SHARED((_N, _L), jnp.float32),
            pltpu.VMEM((4, 2, _K), jnp.int32),
            pltpu.VMEM((3, _K, _D), jnp.float32),
            pltpu.VMEM((3, _K, _L), jnp.float32),
            pltpu.VMEM((3, _K), jnp.float32),
            pltpu.VMEM((3, _K), jnp.float32),
            pltpu.VMEM((_K,), jnp.float32),
            pltpu.SemaphoreType.DMA,
            pltpu.SemaphoreType.DMA,
            pltpu.SemaphoreType.DMA,
            pltpu.SemaphoreType.DMA,
            pltpu.SemaphoreType.DMA,
            pltpu.SemaphoreType.DMA,
            pltpu.SemaphoreType.DMA,
            pltpu.SemaphoreType.DMA,
            pltpu.SemaphoreType.DMA,
            pltpu.SemaphoreType.DMA,
        ],
        compiler_params=pltpu.CompilerParams(use_tc_tiling_on_sc=False,
                                             needs_layout_passes=False),
    )(_sc_body)


def kernel(src_feat, dst_feat, edge_index, n_dst, W, b):
    wt = W.reshape(1, 2 * _D)
    bb = b.reshape(1, 1)
    s1, s2 = pl.pallas_call(
        _prologue_body,
        out_shape=[
            jax.ShapeDtypeStruct((_N, 1), jnp.float32),
            jax.ShapeDtypeStruct((_N, 1), jnp.float32),
        ],
    )(src_feat, dst_feat, wt, bb)
    s1 = s1.reshape(_N)
    s2 = s2.reshape(_N)
    aggp, cntp = _make_sc_kernel()(src_feat, s1, s2, edge_index)
    zr = (jnp.asarray(n_dst, jnp.float32) - jnp.float32(_N)).reshape(1, 1)
    out = pl.pallas_call(
        _epilogue_body,
        out_shape=jax.ShapeDtypeStruct((_N, _D), jnp.float32),
    )(aggp, cntp, zr)
    return out


# speedup vs baseline: 1.0466x; 1.0017x over previous
import functools

import jax
import jax.numpy as jnp
from jax import lax
from jax.experimental import pallas as pl
from jax.experimental.pallas import tpu as pltpu
from jax.experimental.pallas import tpu_sc as plsc

_N = 10000
_D = 128
_E = 320000
_NC = 2
_NS = 16
_L = 16
_K = 80
_EPT = _E // (_NC * _NS)
_NCH = _EPT // _K
_RSTRIDE = 624
_RWIN = 640


def _prologue_body(src_ref, dst_ref, wt_ref, bb_ref, s1_ref, s2_ref):
    w = wt_ref[...]
    w1 = w[:, :_D]
    w2 = w[:, _D:]
    s1_ref[...] = jnp.sum(src_ref[...] * w1, axis=1, keepdims=True)
    s2_ref[...] = jnp.sum(dst_ref[...] * w2, axis=1, keepdims=True) + bb_ref[...]


def _epilogue_body(aggp_ref, cntp_ref, zr_ref, out_ref):
    zr = zr_ref[...]
    agg = aggp_ref[0] + aggp_ref[1] + zr
    cnt = cntp_ref[0, :, 0:1] + cntp_ref[1, :, 0:1] + zr
    cnt = jnp.clip(cnt, 1e-8, None)
    out_ref[...] = agg / cnt


def _sc_body(srcf, s1h, s2h, eidx_h, aggo, cnto,
             agg_spm, cnt_spm,
             eidx, rows, cntr, s1g, s2g, attv,
             isem0, isem1, isem2, isem3,
             gsem0, gsem1, gsem2, ssem0, ssem1, ssem2):
    c = lax.axis_index("c")
    s = lax.axis_index("s")
    tile_base = (c * _NS + s) * _EPT
    row_base = s * _RSTRIDE
    isems = (isem0, isem1, isem2, isem3)
    gsems = (gsem0, gsem1, gsem2)
    ssems = (ssem0, ssem1, ssem2)

    def _early_idx_load(g, i):
        eb = tile_base + g * _K
        pltpu.async_copy(eidx_h.at[:, pl.ds(eb, _K)], eidx.at[i], isems[i])

    _early_idx_load(0, 0)
    _early_idx_load(1, 1)

    zero16 = jnp.zeros((_L,), jnp.float32)

    def _zs(i, carry):
        r = i // 8
        q = lax.rem(i, 8)
        rows[0, r, pl.ds(q * _L, _L)] = zero16
        return carry

    lax.fori_loop(0, _K * 8, _zs, 0)

    def _zc(i, carry):
        cntr[0, i, :] = zero16
        return carry

    lax.fori_loop(0, _K, _zc, 0)

    off = 0
    while off < _RWIN:
        sz = min(_K, _RWIN - off)
        pltpu.sync_copy(rows.at[0, pl.ds(0, sz)],
                        agg_spm.at[pl.ds(row_base + off, sz)])
        pltpu.sync_copy(cntr.at[0, pl.ds(0, sz)],
                        cnt_spm.at[pl.ds(row_base + off, sz)])
        off += sz

    def _idx_load(g, i):
        eb = tile_base + g * _K
        pltpu.async_copy(eidx_h.at[:, pl.ds(eb, _K)], eidx.at[i], isems[i])

    def _wait_idx(g, i):
        eb = tile_base + g * _K
        pltpu.make_async_copy(eidx_h.at[:, pl.ds(eb, _K)], eidx.at[i],
                              isems[i]).wait()

    def _gather(r, i):
        pltpu.async_copy(srcf.at[eidx.at[i, 0]], rows.at[r], gsems[r])
        pltpu.async_copy(s1h.at[eidx.at[i, 0]], s1g.at[r], gsems[r])
        pltpu.async_copy(s2h.at[eidx.at[i, 1]], s2g.at[r], gsems[r])

    def _wait_gather(r, i):
        pltpu.make_async_copy(srcf.at[eidx.at[i, 0]], rows.at[r],
                              gsems[r]).wait()
        pltpu.make_async_copy(s1h.at[eidx.at[i, 0]], s1g.at[r],
                              gsems[r]).wait()
        pltpu.make_async_copy(s2h.at[eidx.at[i, 1]], s2g.at[r],
                              gsems[r]).wait()

    def _compute(r):
        for v in range(_K // _L):
            sl = pl.ds(v * _L, _L)
            att = 1.0 / (1.0 + jnp.exp(-(s1g[r, sl] + s2g[r, sl])))
            attv[sl] = att

        @plsc.parallel_loop(0, _K, step=1, unroll=4)
        def _edge(e):
            bc = plsc.load_gather(attv, [jnp.full((_L,), 0, jnp.int32) + e])
            cntr[r, e, :] = bc
            for dd in range(_D // _L):
                dsl = pl.ds(dd * _L, _L)
                rows[r, e, dsl] = rows[r, e, dsl] * bc

    def _scatter(r, i):
        pltpu.async_copy(rows.at[r], agg_spm.at[eidx.at[i, 1]], ssems[r],
                         add=True)
        pltpu.async_copy(cntr.at[r], cnt_spm.at[eidx.at[i, 1]], ssems[r],
                         add=True)

    def _wait_scatter(r, i):
        pltpu.make_async_copy(rows.at[r], agg_spm.at[eidx.at[i, 1]],
                              ssems[r]).wait()
        pltpu.make_async_copy(cntr.at[r], cnt_spm.at[eidx.at[i, 1]],
                              ssems[r]).wait()

    def _pstep(g, gm3, gm4, p1, p2, p3):
        if p1:
            _wait_scatter((gm3 + 1) % 3, (gm4 + 2) % 4)
        if p2:
            _idx_load(g + 2, (gm4 + 2) % 4)
        if p3:
            _wait_idx(g + 1, (gm4 + 1) % 4)
            _gather((gm3 + 1) % 3, (gm4 + 1) % 4)
        _wait_gather(gm3, gm4)
        _compute(gm3)
        _scatter(gm3, gm4)

    _wait_idx(0, 0)
    _gather(0, 0)
    plsc.subcore_barrier()
    _pstep(0, 0, 0, False, True, True)
    _pstep(1, 1, 1, False, True, True)

    def _twelve(t, carry):
        for j in range(12):
            g = 12 * t + 2 + j
            _pstep(g, (2 + j) % 3, (2 + j) % 4, True, True, True)
        return carry

    lax.fori_loop(0, (_NCH - 5) // 12, _twelve, 0)

    _pstep(_NCH - 3, (_NCH - 3) % 3, (_NCH - 3) % 4, True, True, True)
    _pstep(_NCH - 2, (_NCH - 2) % 3, (_NCH - 2) % 4, True, False, True)
    _pstep(_NCH - 1, (_NCH - 1) % 3, (_NCH - 1) % 4, True, False, False)
    _wait_scatter((_NCH - 2) % 3, (_NCH - 2) % 4)
    _wait_scatter((_NCH - 1) % 3, (_NCH - 1) % 4)
    plsc.subcore_barrier()

    pltpu.sync_copy(agg_spm.at[pl.ds(row_base, _RWIN)],
                    aggo.at[c, pl.ds(row_base, _RWIN)])
    pltpu.sync_copy(cnt_spm.at[pl.ds(row_base, _RWIN)],
                    cnto.at[c, pl.ds(row_base, _RWIN)])


@functools.cache
def _make_sc_kernel():
    return functools.partial(
        pl.kernel,
        mesh=plsc.VectorSubcoreMesh(core_axis_name="c", subcore_axis_name="s",
                                    num_cores=_NC, num_subcores=_NS),
        out_type=[
            jax.ShapeDtypeStruct((_NC, _N, _D), jnp.float32),
            jax.ShapeDtypeStruct((_NC, _N, _L), jnp.float32),
        ],
        scratch_types=[
            pltpu.VMEM_SHARED((_N, _D), jnp.float32),
            pltpu.VMEM_SHARED((_N, _L), jnp.float32),
            pltpu.VMEM((4, 2, _K), jnp.int32),
            pltpu.VMEM((3, _K, _D), jnp.float32),
            pltpu.VMEM((3, _K, _L), jnp.float32),
            pltpu.VMEM((3, _K), jnp.float32),
            pltpu.VMEM((3, _K), jnp.float32),
            pltpu.VMEM((_K,), jnp.float32),
            pltpu.SemaphoreType.DMA,
            pltpu.SemaphoreType.DMA,
            pltpu.SemaphoreType.DMA,
            pltpu.SemaphoreType.DMA,
            pltpu.SemaphoreType.DMA,
            pltpu.SemaphoreType.DMA,
            pltpu.SemaphoreType.DMA,
            pltpu.SemaphoreType.DMA,
            pltpu.SemaphoreType.DMA,
            pltpu.SemaphoreType.DMA,
        ],
        compiler_params=pltpu.CompilerParams(use_tc_tiling_on_sc=False,
                                             needs_layout_passes=False),
    )(_sc_body)


def kernel(src_feat, dst_feat, edge_index, n_dst, W, b):
    wt = W.reshape(1, 2 * _D)
    bb = b.reshape(1, 1)
    s1, s2 = pl.pallas_call(
        _prologue_body,
        out_shape=[
            jax.ShapeDtypeStruct((_N, 1), jnp.float32),
            jax.ShapeDtypeStruct((_N, 1), jnp.float32),
        ],
    )(src_feat, dst_feat, wt, bb)
    s1 = s1.reshape(_N)
    s2 = s2.reshape(_N)
    aggp, cntp = _make_sc_kernel()(src_feat, s1, s2, edge_index)
    zr = (jnp.asarray(n_dst, jnp.float32) - jnp.float32(_N)).reshape(1, 1)
    out = pl.pallas_call(
        _epilogue_body,
        out_shape=jax.ShapeDtypeStruct((_N, _D), jnp.float32),
    )(aggp, cntp, zr)
    return out
